# baseline (speedup 1.0000x reference)
import jax
import jax.numpy as jnp
from jax.experimental import pallas as pl
from jax.experimental.pallas import tpu as pltpu

L1_CAP, L2_CAP, L3_CAP = 2048, 4096, 8192
SDR, CDIM = 2048, 384
N = 2048


_CHUNK_SDR = 256
_CHUNK_C = 1024


def _dma_body(sdrs, contents, l1s, l1c, l2s, l2c, l3s, l3c,
              o1s, o1c, o2s, o2c, o3s, o3c, sem):
    h = N
    plan = [
        (sdrs, o1s, 0, 0, N, _CHUNK_SDR),
        (contents, o1c, 0, 0, N, _CHUNK_C),
        (l1s, o2s, 0, 0, h, _CHUNK_SDR),
        (l1c, o2c, 0, 0, h, _CHUNK_C),
        (l2s, o2s, h, h, L2_CAP - h, _CHUNK_SDR),
        (l2c, o2c, h, h, L2_CAP - h, _CHUNK_C),
        (l2s, o3s, 0, 0, h, _CHUNK_SDR),
        (l2c, o3c, 0, 0, h, _CHUNK_C),
        (l3s, o3s, h, h, L3_CAP - h, _CHUNK_SDR),
        (l3c, o3c, h, h, L3_CAP - h, _CHUNK_C),
    ]
    copies = []
    for src, dst, s0, d0, nrows, ck in plan:
        for r in range(0, nrows, ck):
            m = min(ck, nrows - r)
            copies.append(pltpu.make_async_copy(
                src.at[pl.ds(s0 + r, m)], dst.at[pl.ds(d0 + r, m)],
                sem.at[len(copies)]))
    for c in copies:
        c.start()
    for c in copies:
        c.wait()


def _num_chunks():
    n = 0
    for nrows, ck in [(N, _CHUNK_SDR), (N, _CHUNK_C), (N, _CHUNK_SDR),
                      (N, _CHUNK_C), (L2_CAP - N, _CHUNK_SDR),
                      (L2_CAP - N, _CHUNK_C), (N, _CHUNK_SDR), (N, _CHUNK_C),
                      (L3_CAP - N, _CHUNK_SDR), (L3_CAP - N, _CHUNK_C)]:
        n += -(-nrows // ck)
    return n


def kernel(sdrs, contents, l1_sdr_bank, l1_content_bank,
           l2_sdr_bank, l2_content_bank, l3_sdr_bank, l3_content_bank):
    sdrs = jax.lax.stop_gradient(sdrs)
    contents = jax.lax.stop_gradient(contents)
    out_shape = [
        jax.ShapeDtypeStruct((L1_CAP, SDR), jnp.float32),
        jax.ShapeDtypeStruct((L1_CAP, CDIM), jnp.float32),
        jax.ShapeDtypeStruct((L2_CAP, SDR), jnp.float32),
        jax.ShapeDtypeStruct((L2_CAP, CDIM), jnp.float32),
        jax.ShapeDtypeStruct((L3_CAP, SDR), jnp.float32),
        jax.ShapeDtypeStruct((L3_CAP, CDIM), jnp.float32),
    ]
    any_spec = pl.BlockSpec(memory_space=pl.ANY)
    outs = pl.pallas_call(
        _dma_body,
        out_shape=out_shape,
        in_specs=[any_spec] * 8,
        out_specs=[any_spec] * 6,
        scratch_shapes=[pltpu.SemaphoreType.DMA((_num_chunks(),))],
    )(sdrs, contents, l1_sdr_bank, l1_content_bank,
      l2_sdr_bank, l2_content_bank, l3_sdr_bank, l3_content_bank)
    return tuple(outs)


# speedup vs baseline: 42.5714x; 42.5714x over previous
import functools

import jax
import jax.numpy as jnp
from jax.experimental import pallas as pl
from jax.experimental.pallas import tpu as pltpu

L1_CAP, L2_CAP, L3_CAP = 2048, 4096, 8192
SDR, CDIM = 2048, 384
N = 2048

_BLK = 256


def _copy2_body(a_s, a_c, o_s, o_c):
    o_s[...] = a_s[...]
    o_c[...] = a_c[...]


def _concat_body(split, a_s, a_c, b_s, b_c, o_s, o_c):
    i = pl.program_id(0)

    @pl.when(i < split)
    def _():
        o_s[...] = a_s[...]
        o_c[...] = a_c[...]

    @pl.when(i >= split)
    def _():
        o_s[...] = b_s[...]
        o_c[...] = b_c[...]


def _tier_copy(a_s, a_c):
    rows = a_s.shape[0]
    grid = rows // _BLK
    return pl.pallas_call(
        _copy2_body,
        grid=(grid,),
        in_specs=[
            pl.BlockSpec((_BLK, SDR), lambda i: (i, 0)),
            pl.BlockSpec((_BLK, CDIM), lambda i: (i, 0)),
        ],
        out_specs=[
            pl.BlockSpec((_BLK, SDR), lambda i: (i, 0)),
            pl.BlockSpec((_BLK, CDIM), lambda i: (i, 0)),
        ],
        out_shape=[
            jax.ShapeDtypeStruct((rows, SDR), jnp.float32),
            jax.ShapeDtypeStruct((rows, CDIM), jnp.float32),
        ],
    )(a_s, a_c)


def _tier_concat(a_s, a_c, b_s, b_c, rows, a_rows, b_row0):
    grid = rows // _BLK
    split = a_rows // _BLK
    boff = b_row0 // _BLK

    def a_map(i):
        return (jnp.minimum(i, split - 1), 0)

    def b_map(i):
        return (jnp.maximum(i, split) - split + boff, 0)

    return pl.pallas_call(
        functools.partial(_concat_body, split),
        grid=(grid,),
        in_specs=[
            pl.BlockSpec((_BLK, SDR), a_map),
            pl.BlockSpec((_BLK, CDIM), a_map),
            pl.BlockSpec((_BLK, SDR), b_map),
            pl.BlockSpec((_BLK, CDIM), b_map),
        ],
        out_specs=[
            pl.BlockSpec((_BLK, SDR), lambda i: (i, 0)),
            pl.BlockSpec((_BLK, CDIM), lambda i: (i, 0)),
        ],
        out_shape=[
            jax.ShapeDtypeStruct((rows, SDR), jnp.float32),
            jax.ShapeDtypeStruct((rows, CDIM), jnp.float32),
        ],
    )(a_s, a_c, b_s, b_c)


def kernel(sdrs, contents, l1_sdr_bank, l1_content_bank,
           l2_sdr_bank, l2_content_bank, l3_sdr_bank, l3_content_bank):
    sdrs = jax.lax.stop_gradient(sdrs)
    contents = jax.lax.stop_gradient(contents)

    o1s, o1c = _tier_copy(sdrs, contents)
    o2s, o2c = _tier_concat(l1_sdr_bank, l1_content_bank,
                            l2_sdr_bank, l2_content_bank,
                            rows=L2_CAP, a_rows=N, b_row0=N)
    o3s, o3c = _tier_concat(l2_sdr_bank, l2_content_bank,
                            l3_sdr_bank, l3_content_bank,
                            rows=L3_CAP, a_rows=N, b_row0=N)
    return (o1s, o1c, o2s, o2c, o3s, o3c)


# speedup vs baseline: 45.2101x; 1.0620x over previous
import functools

import jax
import jax.numpy as jnp
from jax.experimental import pallas as pl
from jax.experimental.pallas import tpu as pltpu

L1_CAP, L2_CAP, L3_CAP = 2048, 4096, 8192
SDR, CDIM = 2048, 384
N = 2048

_BLK = 512


def _copy2_body(a_s, a_c, o_s, o_c):
    o_s[...] = a_s[...]
    o_c[...] = a_c[...]


def _concat_body(split, a_s, a_c, b_s, b_c, o_s, o_c):
    i = pl.program_id(0)

    @pl.when(i < split)
    def _():
        o_s[...] = a_s[...]
        o_c[...] = a_c[...]

    @pl.when(i >= split)
    def _():
        o_s[...] = b_s[...]
        o_c[...] = b_c[...]


def _tier_copy(a_s, a_c):
    rows = a_s.shape[0]
    grid = rows // _BLK
    return pl.pallas_call(
        _copy2_body,
        grid=(grid,),
        in_specs=[
            pl.BlockSpec((_BLK, SDR), lambda i: (i, 0)),
            pl.BlockSpec((_BLK, CDIM), lambda i: (i, 0)),
        ],
        out_specs=[
            pl.BlockSpec((_BLK, SDR), lambda i: (i, 0)),
            pl.BlockSpec((_BLK, CDIM), lambda i: (i, 0)),
        ],
        out_shape=[
            jax.ShapeDtypeStruct((rows, SDR), jnp.float32),
            jax.ShapeDtypeStruct((rows, CDIM), jnp.float32),
        ],
    )(a_s, a_c)


def _tier_concat(a_s, a_c, b_s, b_c, rows, a_rows, b_row0):
    grid = rows // _BLK
    split = a_rows // _BLK
    boff = b_row0 // _BLK

    def a_map(i):
        return (jnp.minimum(i, split - 1), 0)

    def b_map(i):
        return (jnp.maximum(i, split) - split + boff, 0)

    return pl.pallas_call(
        functools.partial(_concat_body, split),
        grid=(grid,),
        in_specs=[
            pl.BlockSpec((_BLK, SDR), a_map),
            pl.BlockSpec((_BLK, CDIM), a_map),
            pl.BlockSpec((_BLK, SDR), b_map),
            pl.BlockSpec((_BLK, CDIM), b_map),
        ],
        out_specs=[
            pl.BlockSpec((_BLK, SDR), lambda i: (i, 0)),
            pl.BlockSpec((_BLK, CDIM), lambda i: (i, 0)),
        ],
        out_shape=[
            jax.ShapeDtypeStruct((rows, SDR), jnp.float32),
            jax.ShapeDtypeStruct((rows, CDIM), jnp.float32),
        ],
    )(a_s, a_c, b_s, b_c)


def kernel(sdrs, contents, l1_sdr_bank, l1_content_bank,
           l2_sdr_bank, l2_content_bank, l3_sdr_bank, l3_content_bank):
    sdrs = jax.lax.stop_gradient(sdrs)
    contents = jax.lax.stop_gradient(contents)

    o1s, o1c = _tier_copy(sdrs, contents)
    o2s, o2c = _tier_concat(l1_sdr_bank, l1_content_bank,
                            l2_sdr_bank, l2_content_bank,
                            rows=L2_CAP, a_rows=N, b_row0=N)
    o3s, o3c = _tier_concat(l2_sdr_bank, l2_content_bank,
                            l3_sdr_bank, l3_content_bank,
                            rows=L3_CAP, a_rows=N, b_row0=N)
    return (o1s, o1c, o2s, o2c, o3s, o3c)


# speedup vs baseline: 48.5309x; 1.0735x over previous
import functools

import jax
import jax.numpy as jnp
from jax.experimental import pallas as pl
from jax.experimental.pallas import tpu as pltpu

L1_CAP, L2_CAP, L3_CAP = 2048, 4096, 8192
SDR, CDIM = 2048, 384
N = 2048

_BLK = 1024


def _copy2_body(a_s, a_c, o_s, o_c):
    o_s[...] = a_s[...]
    o_c[...] = a_c[...]


def _concat_body(split, a_s, a_c, b_s, b_c, o_s, o_c):
    i = pl.program_id(0)

    @pl.when(i < split)
    def _():
        o_s[...] = a_s[...]
        o_c[...] = a_c[...]

    @pl.when(i >= split)
    def _():
        o_s[...] = b_s[...]
        o_c[...] = b_c[...]


def _tier_copy(a_s, a_c):
    rows = a_s.shape[0]
    grid = rows // _BLK
    return pl.pallas_call(
        _copy2_body,
        grid=(grid,),
        in_specs=[
            pl.BlockSpec((_BLK, SDR), lambda i: (i, 0)),
            pl.BlockSpec((_BLK, CDIM), lambda i: (i, 0)),
        ],
        out_specs=[
            pl.BlockSpec((_BLK, SDR), lambda i: (i, 0)),
            pl.BlockSpec((_BLK, CDIM), lambda i: (i, 0)),
        ],
        out_shape=[
            jax.ShapeDtypeStruct((rows, SDR), jnp.float32),
            jax.ShapeDtypeStruct((rows, CDIM), jnp.float32),
        ],
    )(a_s, a_c)


def _tier_concat(a_s, a_c, b_s, b_c, rows, a_rows, b_row0):
    grid = rows // _BLK
    split = a_rows // _BLK
    boff = b_row0 // _BLK

    def a_map(i):
        return (jnp.minimum(i, split - 1), 0)

    def b_map(i):
        return (jnp.maximum(i, split) - split + boff, 0)

    return pl.pallas_call(
        functools.partial(_concat_body, split),
        grid=(grid,),
        in_specs=[
            pl.BlockSpec((_BLK, SDR), a_map),
            pl.BlockSpec((_BLK, CDIM), a_map),
            pl.BlockSpec((_BLK, SDR), b_map),
            pl.BlockSpec((_BLK, CDIM), b_map),
        ],
        out_specs=[
            pl.BlockSpec((_BLK, SDR), lambda i: (i, 0)),
            pl.BlockSpec((_BLK, CDIM), lambda i: (i, 0)),
        ],
        out_shape=[
            jax.ShapeDtypeStruct((rows, SDR), jnp.float32),
            jax.ShapeDtypeStruct((rows, CDIM), jnp.float32),
        ],
    )(a_s, a_c, b_s, b_c)


def kernel(sdrs, contents, l1_sdr_bank, l1_content_bank,
           l2_sdr_bank, l2_content_bank, l3_sdr_bank, l3_content_bank):
    sdrs = jax.lax.stop_gradient(sdrs)
    contents = jax.lax.stop_gradient(contents)

    o1s, o1c = _tier_copy(sdrs, contents)
    o2s, o2c = _tier_concat(l1_sdr_bank, l1_content_bank,
                            l2_sdr_bank, l2_content_bank,
                            rows=L2_CAP, a_rows=N, b_row0=N)
    o3s, o3c = _tier_concat(l2_sdr_bank, l2_content_bank,
                            l3_sdr_bank, l3_content_bank,
                            rows=L3_CAP, a_rows=N, b_row0=N)
    return (o1s, o1c, o2s, o2c, o3s, o3c)
